# baseline (speedup 1.0000x reference)
import jax
import jax.numpy as jnp
from jax.experimental import pallas as pl
from jax.experimental.pallas import tpu as pltpu


def _copy_body(idx_ref, in_ref, out_ref):
    out_ref[...] = in_ref[...]


def _gather_views(img4, indices, n):
    B, V, M, L = img4.shape
    return pl.pallas_call(
        _copy_body,
        grid_spec=pltpu.PrefetchScalarGridSpec(
            num_scalar_prefetch=1,
            grid=(B, n),
            in_specs=[pl.BlockSpec((1, 1, M, L), lambda b, v, idx: (b, idx[b, v], 0, 0))],
            out_specs=pl.BlockSpec((1, 1, M, L), lambda b, v, idx: (b, v, 0, 0)),
        ),
        out_shape=jax.ShapeDtypeStruct((B, n, M, L), img4.dtype),
    )(indices, img4)


def kernel(image, context_indices, target_indices):
    B, V, C, H, W = image.shape
    D = C * H * W
    L = 128
    img4 = image.reshape(B, V, D // L, L)
    n_in = context_indices.shape[1]
    n_tg = target_indices.shape[1]
    input_image = _gather_views(img4, context_indices, n_in).reshape(B, n_in, C, H, W)
    target_image = _gather_views(img4, target_indices, n_tg).reshape(B, n_tg, C, H, W)
    return (input_image, target_image, context_indices, target_indices)


# speedup vs baseline: 1.0872x; 1.0872x over previous
import functools

import jax
import jax.numpy as jnp
from jax import lax
from jax.experimental import pallas as pl
from jax.experimental.pallas import tpu as pltpu
from jax.experimental.pallas import tpu_sc as plsc

_NC, _NS, _L = 2, 16, 16
_NW = _NC * _NS
_S = 49
_CHUNK = 3072
_G = 16


def _sc_split(img2d, grp_in, grp_tg):
    G_in = grp_in.shape[0]
    G_tg = grp_tg.shape[0]
    mesh = plsc.VectorSubcoreMesh(core_axis_name="c", subcore_axis_name="s")

    @functools.partial(
        pl.kernel,
        out_type=[
            jax.ShapeDtypeStruct((G_in * _G, _CHUNK), jnp.float32),
            jax.ShapeDtypeStruct((G_tg * _G, _CHUNK), jnp.float32),
        ],
        mesh=mesh,
        scratch_types=[
            pltpu.VMEM((_G,), jnp.int32),
            pltpu.VMEM((_G, _CHUNK), jnp.float32),
            pltpu.SemaphoreType.DMA,
        ],
    )
    def k(img_hbm, gi_hbm, gt_hbm, out_in, out_tg, idx_v, buf, sem):
        wid = lax.axis_index("s") * _NC + lax.axis_index("c")

        def do_phase(n_groups, grp_hbm, out_hbm):
            def body(i, carry):
                g = wid + i * _NW

                @pl.when(g < n_groups)
                def _():
                    pltpu.sync_copy(grp_hbm.at[g], idx_v)
                    pltpu.async_copy(img_hbm.at[idx_v], buf, sem).wait()
                    pltpu.sync_copy(buf, out_hbm.at[pl.ds(g * _G, _G)])

                return carry

            niter = (n_groups + _NW - 1) // _NW
            lax.fori_loop(0, niter, body, 0)

        do_phase(G_in, gi_hbm, out_in)
        do_phase(G_tg, gt_hbm, out_tg)

    return k(img2d, grp_in, grp_tg)


def _src_row_groups(indices, B, V, n):
    base = (jnp.arange(B, dtype=jnp.int32)[:, None] * V + indices) * _S
    rows = base[:, :, None] + jnp.arange(_S, dtype=jnp.int32)[None, None, :]
    return rows.reshape(-1, _G)


def kernel(image, context_indices, target_indices):
    B, V, C, H, W = image.shape
    n_in = context_indices.shape[1]
    n_tg = target_indices.shape[1]
    img2d = image.reshape(B * V * _S, _CHUNK)
    grp_in = _src_row_groups(context_indices, B, V, n_in)
    grp_tg = _src_row_groups(target_indices, B, V, n_tg)
    out_in, out_tg = _sc_split(img2d, grp_in, grp_tg)
    input_image = out_in.reshape(B, n_in, C, H, W)
    target_image = out_tg.reshape(B, n_tg, C, H, W)
    return (input_image, target_image, context_indices, target_indices)
